# baseline (speedup 1.0000x reference)
import functools

import jax
import jax.numpy as jnp
from jax import lax
from jax.experimental import pallas as pl
from jax.experimental.pallas import tpu as pltpu
from jax.experimental.pallas import tpu_sc as plsc

B = 16384
F = 26
V = 1000000

NC = 2
NS = 16
NW = NC * NS
BPW = B // NW
LANES = 16
NCHUNK = BPW // LANES
VP = V // 2


def _build_sc_call():
    mesh = plsc.VectorSubcoreMesh(core_axis_name="c", subcore_axis_name="s")

    @functools.partial(
        pl.kernel,
        mesh=mesh,
        compiler_params=pltpu.CompilerParams(
            needs_layout_passes=False,
            use_tc_tiling_on_sc=False,
            skip_device_barrier=True,
        ),
        out_type=jax.ShapeDtypeStruct((B,), jnp.float32),
        scratch_types=[
            pltpu.VMEM((F, BPW), jnp.int32),
            pltpu.VMEM((F, BPW), jnp.int32),
            pltpu.VMEM((F, BPW), jnp.int32),
            pltpu.VMEM((BPW,), jnp.float32),
            pltpu.SemaphoreType.DMA,
        ],
    )
    def sc_body(*refs):
        w_refs = refs[:F]
        x_hbm, out_hbm, x_v, idx_v, buf, acc_v, sem = refs[F:]
        wid = lax.axis_index("s") * NC + lax.axis_index("c")

        pltpu.sync_copy(x_hbm.at[wid], x_v)

        for f in range(F):
            def ibody(i, carry, f=f):
                xv = x_v[f, pl.ds(i * LANES, LANES)]
                idx_v[f, pl.ds(i * LANES, LANES)] = xv >> 1
                return carry

            lax.fori_loop(0, NCHUNK, ibody, 0)
            pltpu.async_copy(w_refs[f].at[idx_v.at[f]], buf.at[f], sem)

        for f in range(F):
            pltpu.make_async_copy(
                w_refs[f].at[idx_v.at[f]], buf.at[f], sem
            ).wait()

        def rbody(c, carry):
            sl = pl.ds(c * LANES, LANES)
            acc = jnp.zeros((LANES,), jnp.float32)
            for f in range(F):
                pv = buf[f, sl]
                xv = x_v[f, sl]
                sh = (xv & 1) << 4
                bits = lax.shift_left(
                    lax.shift_right_logical(pv, sh) & 0xFFFF, 16
                )
                acc = acc + lax.bitcast_convert_type(bits, jnp.float32)
            acc_v[sl] = 1.0 / (1.0 + jnp.exp(-acc))
            return carry

        lax.fori_loop(0, NCHUNK, rbody, 0)

        pltpu.sync_copy(acc_v, out_hbm.at[pl.ds(wid * BPW, BPW)])

    return sc_body


_sc_call = _build_sc_call()


@jax.jit
def kernel(X, W_linear):
    x3 = X.T.reshape(F, NW, BPW).transpose(1, 0, 2)
    w_rows = [
        lax.bitcast_convert_type(
            W_linear[f].astype(jnp.bfloat16).reshape(VP, 2), jnp.int32
        )
        for f in range(F)
    ]
    out = _sc_call(*w_rows, x3)
    return out.reshape(B, 1)


# speedup vs baseline: 16.4443x; 16.4443x over previous
import functools

import jax
import jax.numpy as jnp
from jax import lax
from jax.experimental import pallas as pl
from jax.experimental.pallas import tpu as pltpu
from jax.experimental.pallas import tpu_sc as plsc

B = 16384
F = 26
V = 1000000

NC = 2
NS = 16
NW = NC * NS
BPW = B // NW
LANES = 16
NCHUNK = BPW // LANES
NG = (F + 7) // 8


def _build_sc_call():
    mesh = plsc.VectorSubcoreMesh(core_axis_name="c", subcore_axis_name="s")

    @functools.partial(
        pl.kernel,
        mesh=mesh,
        compiler_params=pltpu.CompilerParams(
            needs_layout_passes=False,
            use_tc_tiling_on_sc=False,
            skip_device_barrier=True,
        ),
        out_type=jax.ShapeDtypeStruct((B,), jnp.float32),
        scratch_types=[
            pltpu.VMEM((F, BPW), jnp.int32),
            pltpu.VMEM((F, BPW), jnp.float32),
            pltpu.VMEM((BPW,), jnp.float32),
            pltpu.SemaphoreType.DMA,
        ],
    )
    def sc_body(*refs):
        w_refs = refs[:F]
        x_hbm, out_hbm, x_v, buf, acc_v, sem = refs[F:]
        wid = lax.axis_index("s") * NC + lax.axis_index("c")

        pltpu.sync_copy(x_hbm.at[wid], x_v)

        for f in range(F):
            pltpu.async_copy(w_refs[f].at[x_v.at[f]], buf.at[f], sem)

        for f in range(F):
            pltpu.make_async_copy(
                w_refs[f].at[x_v.at[f]], buf.at[f], sem
            ).wait()

        def rbody(c, carry):
            acc = buf[0, pl.ds(c * LANES, LANES)]
            for f in range(1, F):
                acc = acc + buf[f, pl.ds(c * LANES, LANES)]
            acc_v[pl.ds(c * LANES, LANES)] = 1.0 / (1.0 + jnp.exp(-acc))
            return carry

        lax.fori_loop(0, NCHUNK, rbody, 0)

        pltpu.sync_copy(acc_v, out_hbm.at[pl.ds(wid * BPW, BPW)])

    return sc_body


_sc_call = _build_sc_call()


@jax.jit
def kernel(X, W_linear):
    x3 = X.T.reshape(F, NW, BPW).transpose(1, 0, 2)
    w_rows = [W_linear[f] for f in range(F)]
    out = _sc_call(*w_rows, x3)
    return out.reshape(B, 1)
